# baseline (speedup 1.0000x reference)
import math

import jax
import jax.numpy as jnp
from jax.experimental import pallas as pl
from jax.experimental.pallas import tpu as pltpu

_VMEM_LIMIT = 100 * 1024 * 1024
_BF = jnp.bfloat16


def _dot(a, b):
    return jnp.dot(a, b, preferred_element_type=jnp.float32)


def _relu(v):
    return jnp.maximum(v, 0.0)


def _fused_kernel(x_ref, w1_ref, c2_ref, wpk_ref, bpk_ref,
                  we_ref, wd2_ref, bd2_ref,
                  c1_ref, dw2_ref,
                  dec_ref, ldj_ref, lpz_ref,
                  h1_scr, he_scr, ho_scr, d2p_scr, hdp_scr,
                  w2e_scr, w2o_scr, w1b_scr,
                  we_scr, wd2_scr, we_sem, wd2_sem):
    K = w1_ref.shape[0]
    P = K - 1
    Ho1 = h1_scr.shape[0]
    Ho2 = he_scr.shape[0]
    Hd2 = dec_ref.shape[0]
    B = x_ref.shape[1]
    NF1, NF2 = 32, 64

    wml = wpk_ref[0:128, 0:256]
    ws1 = wpk_ref[0:128, 256:512]
    ws2 = wpk_ref[0:256, 512:768]
    ws3 = wpk_ref[0:256, 768:1024]
    wp = wpk_ref[0:256, 1024:1152]
    wd1 = wpk_ref[0:128, 1152:1280]
    b2h = bpk_ref[:, 0:640]
    be = bpk_ref[:, 1280:1408]
    bml = bpk_ref[:, 1408:1664]
    bs1 = bpk_ref[:, 1664:1920]
    bs2 = bpk_ref[:, 1920:2176]
    bs3 = bpk_ref[:, 2176:2432]
    bp = bpk_ref[:, 2432:2560]
    bd1 = bpk_ref[:, 2560:2688]
    b1 = bpk_ref[:, 2688:3392]
    db1 = bpk_ref[:, 3456:4160]
    db2 = bpk_ref[:, 4224:4296]

    wd2_cp = pltpu.make_async_copy(wd2_ref, wd2_scr, wd2_sem)
    wd2_cp.start()
    we_cp = pltpu.make_async_copy(we_ref, we_scr, we_sem)
    we_cp.start()

    w2e_scr[...] = jnp.zeros_like(w2e_scr)
    w2o_scr[...] = jnp.zeros_like(w2o_scr)
    for j in range(Ho2 // 2):
        w2e_scr[:, (2 * j) * NF1:(2 * j + K) * NF1, j * NF2:(j + 1) * NF2] = \
            c2_ref[:, :, (2 * j) * NF2:(2 * j + 1) * NF2].astype(_BF)
        w2o_scr[:, (2 * j + 1) * NF1:(2 * j + 1 + K) * NF1, j * NF2:(j + 1) * NF2] = \
            c2_ref[:, :, (2 * j + 1) * NF2:(2 * j + 2) * NF2].astype(_BF)
    w1b_scr[...] = jnp.zeros_like(w1b_scr)
    for o in range(Ho1):
        so = min(max(o - 2, 0), Ho2 - K) * NF2
        w1b_scr[:, so:so + K * NF2, o * NF1:(o + 1) * NF1] = \
            c1_ref[:, :, o * NF1:(o + 1) * NF1].astype(_BF)


    acc = _dot(x_ref[0:Ho1].reshape(Ho1 * B, -1), w1_ref[0])
    for di in range(1, K):
        acc = acc + _dot(x_ref[di:di + Ho1].reshape(Ho1 * B, -1), w1_ref[di])
    h1_scr[...] = _relu(acc + b1).astype(_BF).reshape(h1_scr.shape)

    acc_e = _dot(h1_scr[0:Ho2].reshape(Ho2 * B, -1), w2e_scr[0])
    acc_o = _dot(h1_scr[0:Ho2].reshape(Ho2 * B, -1), w2o_scr[0])
    for di in range(1, K):
        lhs = h1_scr[di:di + Ho2].reshape(Ho2 * B, -1)
        acc_e = acc_e + _dot(lhs, w2e_scr[di])
        acc_o = acc_o + _dot(lhs, w2o_scr[di])
    he_scr[...] = (acc_e + b2h).reshape(he_scr.shape)
    ho_scr[...] = (acc_o + b2h).reshape(ho_scr.shape)

    pieces = []
    for i in range(Ho2 // 2):
        pieces.append(_relu(jnp.maximum(
            jnp.maximum(he_scr[2 * i], he_scr[2 * i + 1]),
            jnp.maximum(ho_scr[2 * i], ho_scr[2 * i + 1]))))
    hf = jnp.concatenate(pieces, axis=1)

    L = 128

    we_cp.wait()
    h = _relu(_dot(hf, we_scr[...]) + be)
    ml = _dot(h, wml) + bml
    xa, xb = ml[:, :L], ml[:, L:]

    st = _relu(_dot(xa, ws1) + bs1)
    st = _relu(_dot(st, ws2) + bs2)
    st = _dot(st, ws3) + bs3
    s = _relu(st[:, :L])
    t = st[:, L:]

    yb = (xb - t) * jnp.exp(-s)
    ldj_ref[...] = -jnp.sum(s, axis=1, keepdims=True)

    z = _relu(_dot(xa, wp[:L, :]) + _dot(yb, wp[L:, :]) + bp)
    lpz_ref[...] = (-0.5 * jnp.sum(z * z, axis=1, keepdims=True)
                    - 0.5 * z.shape[1] * math.log(2.0 * math.pi))

    d = _relu(_dot(z, wd1) + bd1)
    wd2_cp.wait()
    d2 = _relu(_dot(d, wd2_scr[...]) + bd2_ref[...])

    WCi = w1b_scr.shape[1]
    d2p_scr[0] = jnp.zeros_like(d2p_scr[0])
    d2p_scr[1] = jnp.zeros_like(d2p_scr[1])
    d2p_scr[Ho2 + P] = jnp.zeros_like(d2p_scr[0])
    d2p_scr[Ho2 + P + 1] = jnp.zeros_like(d2p_scr[0])
    for r in range(Ho2):
        d2p_scr[r + P] = d2[:, r * WCi:(r + 1) * WCi].astype(_BF)

    Hd1 = Ho1
    acc = _dot(d2p_scr[0:Hd1].reshape(Hd1 * B, -1), w1b_scr[0])
    for di in range(1, K):
        acc = acc + _dot(d2p_scr[di:di + Hd1].reshape(Hd1 * B, -1), w1b_scr[di])
    hd = _relu(acc + db1)
    hdp_scr[0] = jnp.zeros_like(hdp_scr[0])
    hdp_scr[1] = jnp.zeros_like(hdp_scr[1])
    hdp_scr[Hd1 + P] = jnp.zeros_like(hdp_scr[0])
    hdp_scr[Hd1 + P + 1] = jnp.zeros_like(hdp_scr[0])
    hdp_scr[P:Hd1 + P] = hd.reshape(Hd1, B, -1)

    acc = _dot(hdp_scr[0:Hd2].reshape(Hd2 * B, -1), dw2_ref[0])
    for di in range(1, K):
        acc = acc + _dot(hdp_scr[di:di + Hd2].reshape(Hd2 * B, -1), dw2_ref[di])
    dec_ref[...] = (acc + db2).reshape(dec_ref.shape)


def kernel(enc_r1, enc_w1, enc_b1, enc_r2, enc_w2, enc_b2,
           pool_re, pool_ro, pool_ce, pool_co,
           fc_we, fc_be, fc_wml, fc_bml, fc_wst1, fc_bst1,
           fc_wst2, fc_bst2, fc_wst3, fc_bst3, fc_wp, fc_bp,
           fc_wd1, fc_bd1, fc_wd2, fc_bd2,
           dec_r1, dec_w1, dec_b1, dec_r2, dec_w2, dec_b2,
           x_nchw):
    B, C, H, W = x_nchw.shape
    K = enc_w1.shape[0]
    Ho1 = H - K + 1
    Ho2 = Ho1 - K + 1
    D_out = fc_wd2.shape[1]

    xT = jnp.transpose(x_nchw.astype(jnp.float32), (2, 0, 3, 1)).reshape(H, B, W * C)

    NF1, NF2 = 32, 64
    c2 = jnp.concatenate(
        [enc_w2[:, w * NF1:(w + K) * NF1, w * NF2:(w + 1) * NF2]
         for w in range(Ho2)], axis=2)
    c1 = jnp.concatenate(
        [dec_w1[:, min(max(o - 2, 0), Ho2 - K) * NF2:
                (min(max(o - 2, 0), Ho2 - K) + K) * NF2, o * NF1:(o + 1) * NF1]
         for o in range(Ho1)], axis=2)

    pad128 = lambda w: jnp.concatenate(
        [w, jnp.zeros((256 - w.shape[0], w.shape[1]), jnp.float32)], axis=0)
    wpk = jnp.concatenate(
        [pad128(fc_wml), pad128(fc_wst1), fc_wst2, fc_wst3,
         fc_wp, pad128(fc_wd1)], axis=1)
    zl = lambda n: jnp.zeros((1, n), jnp.float32)
    bpk = jnp.concatenate(
        [enc_b2[:, :640], zl(640), fc_be, fc_bml, fc_bst1, fc_bst2, fc_bst3,
         fc_bp, fc_bd1, enc_b1, zl(64), dec_b1, zl(64), dec_b2, zl(56)],
        axis=1)
    ws = (enc_w1, c2, wpk, bpk, fc_we, fc_wd2, fc_bd2, c1, dec_w2)

    vm = pl.BlockSpec(memory_space=pltpu.MemorySpace.VMEM)
    hbm = pl.BlockSpec(memory_space=pl.ANY)
    in_specs = [vm] * (1 + len(ws))
    in_specs[1 + 4] = hbm
    in_specs[1 + 5] = hbm
    dec, ldj, lpz = pl.pallas_call(
        _fused_kernel,
        out_shape=(jax.ShapeDtypeStruct((H, B, W * C), jnp.float32),
                   jax.ShapeDtypeStruct((B, 1), jnp.float32),
                   jax.ShapeDtypeStruct((B, 1), jnp.float32)),
        in_specs=in_specs,
        out_specs=(vm, vm, vm),
        scratch_shapes=[
            pltpu.VMEM((Ho1, B, enc_w1.shape[2]), _BF),
            pltpu.VMEM((Ho2, B, enc_w2.shape[2] // 2), jnp.float32),
            pltpu.VMEM((Ho2, B, enc_w2.shape[2] // 2), jnp.float32),
            pltpu.VMEM((Ho2 + 2 * (K - 1), B, dec_w1.shape[1]), _BF),
            pltpu.VMEM((Ho1 + 2 * (K - 1), B, dec_w1.shape[2]), jnp.float32),
            pltpu.VMEM((enc_w2.shape[0], enc_w2.shape[1], enc_w2.shape[2] // 2), _BF),
            pltpu.VMEM((enc_w2.shape[0], enc_w2.shape[1], enc_w2.shape[2] // 2), _BF),
            pltpu.VMEM(dec_w1.shape, _BF),
            pltpu.VMEM(fc_we.shape, jnp.float32),
            pltpu.VMEM(fc_wd2.shape, jnp.float32),
            pltpu.SemaphoreType.DMA,
            pltpu.SemaphoreType.DMA,
        ],
        compiler_params=pltpu.CompilerParams(vmem_limit_bytes=_VMEM_LIMIT),
    )(xT, *ws)

    x_hat = jnp.transpose(dec.reshape(H, B, W, C), (1, 3, 0, 2))
    return x_hat, ldj[:, 0], lpz[:, 0]


# speedup vs baseline: 1.0305x; 1.0305x over previous
import math

import jax
import jax.numpy as jnp
from jax.experimental import pallas as pl
from jax.experimental.pallas import tpu as pltpu

_VMEM_LIMIT = 100 * 1024 * 1024
_BF = jnp.bfloat16


def _dot(a, b):
    return jnp.dot(a, b, preferred_element_type=jnp.float32)


def _relu(v):
    return jnp.maximum(v, 0.0)


def _fused_kernel(x_ref, w1_ref, b1_ref, c2_ref, b2_ref,
                  we_ref, be_ref, wml_ref, bml_ref,
                  ws1_ref, bs1_ref, ws2_ref, bs2_ref, ws3_ref, bs3_ref,
                  wp_ref, bp_ref, wd1_ref, bd1_ref, wd2_ref, bd2_ref,
                  c1_ref, db1_ref, dw2_ref, db2_ref,
                  dec_ref, ldj_ref, lpz_ref,
                  h1_scr, he_scr, ho_scr, d2p_scr, hdp_scr,
                  w2e_scr, w2o_scr, w1b_scr,
                  we_scr, wd2_scr, we_sem, wd2_sem):
    K = w1_ref.shape[0]
    P = K - 1
    Ho1 = h1_scr.shape[0]
    Ho2 = he_scr.shape[0]
    Hd2 = dec_ref.shape[0]
    B = x_ref.shape[1]
    NF1, NF2 = 32, 64

    wd2_cp = pltpu.make_async_copy(wd2_ref, wd2_scr, wd2_sem)
    wd2_cp.start()
    we_cp = pltpu.make_async_copy(we_ref, we_scr, we_sem)
    we_cp.start()

    w2e_scr[...] = jnp.zeros_like(w2e_scr)
    w2o_scr[...] = jnp.zeros_like(w2o_scr)
    for j in range(Ho2 // 2):
        w2e_scr[:, (2 * j) * NF1:(2 * j + K) * NF1, j * NF2:(j + 1) * NF2] = \
            c2_ref[:, :, (2 * j) * NF2:(2 * j + 1) * NF2].astype(_BF)
        w2o_scr[:, (2 * j + 1) * NF1:(2 * j + 1 + K) * NF1, j * NF2:(j + 1) * NF2] = \
            c2_ref[:, :, (2 * j + 1) * NF2:(2 * j + 2) * NF2].astype(_BF)
    w1b_scr[...] = jnp.zeros_like(w1b_scr)
    for o in range(Ho1):
        so = min(max(o - 2, 0), Ho2 - K) * NF2
        w1b_scr[:, so:so + K * NF2, o * NF1:(o + 1) * NF1] = \
            c1_ref[:, :, o * NF1:(o + 1) * NF1].astype(_BF)


    acc = _dot(x_ref[0:Ho1].reshape(Ho1 * B, -1), w1_ref[0])
    for di in range(1, K):
        acc = acc + _dot(x_ref[di:di + Ho1].reshape(Ho1 * B, -1), w1_ref[di])
    h1_scr[...] = _relu(acc + b1_ref[...]).astype(_BF).reshape(h1_scr.shape)

    acc_e = _dot(h1_scr[0:Ho2].reshape(Ho2 * B, -1), w2e_scr[0])
    acc_o = _dot(h1_scr[0:Ho2].reshape(Ho2 * B, -1), w2o_scr[0])
    for di in range(1, K):
        lhs = h1_scr[di:di + Ho2].reshape(Ho2 * B, -1)
        acc_e = acc_e + _dot(lhs, w2e_scr[di])
        acc_o = acc_o + _dot(lhs, w2o_scr[di])
    b2h = b2_ref[:, :he_scr.shape[2]]
    he_scr[...] = (acc_e + b2h).reshape(he_scr.shape)
    ho_scr[...] = (acc_o + b2h).reshape(ho_scr.shape)

    pieces = []
    for i in range(Ho2 // 2):
        pieces.append(_relu(jnp.maximum(
            jnp.maximum(he_scr[2 * i], he_scr[2 * i + 1]),
            jnp.maximum(ho_scr[2 * i], ho_scr[2 * i + 1]))))
    hf = jnp.concatenate(pieces, axis=1)

    L = wml_ref.shape[1] // 2

    we_cp.wait()
    h = _relu(_dot(hf, we_scr[...]) + be_ref[...])
    ml = _dot(h, wml_ref[...]) + bml_ref[...]
    xa, xb = ml[:, :L], ml[:, L:]

    st = _relu(_dot(xa, ws1_ref[...]) + bs1_ref[...])
    st = _relu(_dot(st, ws2_ref[...]) + bs2_ref[...])
    st = _dot(st, ws3_ref[...]) + bs3_ref[...]
    s = _relu(st[:, :L])
    t = st[:, L:]

    yb = (xb - t) * jnp.exp(-s)
    ldj_ref[...] = -jnp.sum(s, axis=1, keepdims=True)

    z = _relu(_dot(xa, wp_ref[:L, :]) + _dot(yb, wp_ref[L:, :]) + bp_ref[...])
    lpz_ref[...] = (-0.5 * jnp.sum(z * z, axis=1, keepdims=True)
                    - 0.5 * z.shape[1] * math.log(2.0 * math.pi))

    d = _relu(_dot(z, wd1_ref[...]) + bd1_ref[...])
    wd2_cp.wait()
    d2 = _relu(_dot(d, wd2_scr[...]) + bd2_ref[...])

    WCi = w1b_scr.shape[1]
    d2p_scr[0] = jnp.zeros_like(d2p_scr[0])
    d2p_scr[1] = jnp.zeros_like(d2p_scr[1])
    d2p_scr[Ho2 + P] = jnp.zeros_like(d2p_scr[0])
    d2p_scr[Ho2 + P + 1] = jnp.zeros_like(d2p_scr[0])
    for r in range(Ho2):
        d2p_scr[r + P] = d2[:, r * WCi:(r + 1) * WCi].astype(_BF)

    Hd1 = Ho1
    acc = _dot(d2p_scr[0:Hd1].reshape(Hd1 * B, -1), w1b_scr[0])
    for di in range(1, K):
        acc = acc + _dot(d2p_scr[di:di + Hd1].reshape(Hd1 * B, -1), w1b_scr[di])
    hd = _relu(acc + db1_ref[...])
    hdp_scr[0] = jnp.zeros_like(hdp_scr[0])
    hdp_scr[1] = jnp.zeros_like(hdp_scr[1])
    hdp_scr[Hd1 + P] = jnp.zeros_like(hdp_scr[0])
    hdp_scr[Hd1 + P + 1] = jnp.zeros_like(hdp_scr[0])
    hdp_scr[P:Hd1 + P] = hd.reshape(Hd1, B, -1)

    acc = _dot(hdp_scr[0:Hd2].reshape(Hd2 * B, -1), dw2_ref[0])
    for di in range(1, K):
        acc = acc + _dot(hdp_scr[di:di + Hd2].reshape(Hd2 * B, -1), dw2_ref[di])
    dec_ref[...] = (acc + db2_ref[...]).reshape(dec_ref.shape)


def kernel(enc_r1, enc_w1, enc_b1, enc_r2, enc_w2, enc_b2,
           pool_re, pool_ro, pool_ce, pool_co,
           fc_we, fc_be, fc_wml, fc_bml, fc_wst1, fc_bst1,
           fc_wst2, fc_bst2, fc_wst3, fc_bst3, fc_wp, fc_bp,
           fc_wd1, fc_bd1, fc_wd2, fc_bd2,
           dec_r1, dec_w1, dec_b1, dec_r2, dec_w2, dec_b2,
           x_nchw):
    B, C, H, W = x_nchw.shape
    K = enc_w1.shape[0]
    Ho1 = H - K + 1
    Ho2 = Ho1 - K + 1
    D_out = fc_wd2.shape[1]

    xT = jnp.transpose(x_nchw.astype(jnp.float32), (2, 0, 3, 1)).reshape(H, B, W * C)

    NF1, NF2 = 32, 64
    c2 = jnp.concatenate(
        [enc_w2[:, w * NF1:(w + K) * NF1, w * NF2:(w + 1) * NF2]
         for w in range(Ho2)], axis=2)
    c1 = jnp.concatenate(
        [dec_w1[:, min(max(o - 2, 0), Ho2 - K) * NF2:
                (min(max(o - 2, 0), Ho2 - K) + K) * NF2, o * NF1:(o + 1) * NF1]
         for o in range(Ho1)], axis=2)

    ws = (enc_w1, enc_b1, c2, enc_b2,
          fc_we, fc_be, fc_wml, fc_bml, fc_wst1, fc_bst1,
          fc_wst2, fc_bst2, fc_wst3, fc_bst3, fc_wp, fc_bp,
          fc_wd1, fc_bd1, fc_wd2, fc_bd2,
          c1, dec_b1, dec_w2, dec_b2)

    vm = pl.BlockSpec(memory_space=pltpu.MemorySpace.VMEM)
    hbm = pl.BlockSpec(memory_space=pl.ANY)
    in_specs = [vm] * (1 + len(ws))
    in_specs[1 + 4] = hbm
    in_specs[1 + 18] = hbm
    dec, ldj, lpz = pl.pallas_call(
        _fused_kernel,
        out_shape=(jax.ShapeDtypeStruct((H, B, W * C), jnp.float32),
                   jax.ShapeDtypeStruct((B, 1), jnp.float32),
                   jax.ShapeDtypeStruct((B, 1), jnp.float32)),
        in_specs=in_specs,
        out_specs=(vm, vm, vm),
        scratch_shapes=[
            pltpu.VMEM((Ho1, B, enc_w1.shape[2]), _BF),
            pltpu.VMEM((Ho2, B, enc_w2.shape[2] // 2), jnp.float32),
            pltpu.VMEM((Ho2, B, enc_w2.shape[2] // 2), jnp.float32),
            pltpu.VMEM((Ho2 + 2 * (K - 1), B, dec_w1.shape[1]), _BF),
            pltpu.VMEM((Ho1 + 2 * (K - 1), B, dec_w1.shape[2]), jnp.float32),
            pltpu.VMEM((enc_w2.shape[0], enc_w2.shape[1], enc_w2.shape[2] // 2), _BF),
            pltpu.VMEM((enc_w2.shape[0], enc_w2.shape[1], enc_w2.shape[2] // 2), _BF),
            pltpu.VMEM(dec_w1.shape, _BF),
            pltpu.VMEM(fc_we.shape, jnp.float32),
            pltpu.VMEM(fc_wd2.shape, jnp.float32),
            pltpu.SemaphoreType.DMA,
            pltpu.SemaphoreType.DMA,
        ],
        compiler_params=pltpu.CompilerParams(vmem_limit_bytes=_VMEM_LIMIT),
    )(xT, *ws)

    x_hat = jnp.transpose(dec.reshape(H, B, W, C), (1, 3, 0, 2))
    return x_hat, ldj[:, 0], lpz[:, 0]
